# baseline (speedup 1.0000x reference)
import jax
import jax.numpy as jnp
import numpy as np
from jax.experimental import pallas as pl
from jax.experimental.pallas import tpu as pltpu
from jax.sharding import Mesh, PartitionSpec as P

try:
    from jax.experimental.shard_map import shard_map as _shard_map
except ImportError:
    _shard_map = jax.shard_map

_M = 512
_DS = 256
_BT = 1024


def _selector_kernel(x_ref, Ws0_ref, bs0_ref, Ws1_ref, bs1_ref, part_ref):
    x = x_ref[...]
    h = jnp.maximum(
        jnp.dot(x, Ws0_ref[...], preferred_element_type=jnp.float32)
        + bs0_ref[...], 0.0)
    s = (jnp.dot(h, Ws1_ref[...], preferred_element_type=jnp.float32)
         + bs1_ref[...])
    e = jnp.exp(-s)
    p = e / jnp.sum(e, axis=1, keepdims=True)
    part_ref[...] = jnp.sum(p, axis=0, keepdims=True).reshape(1, 1, -1)


def _finalize_kernel(part_ref, w_ref):
    w = jnp.sum(part_ref[...], axis=0)
    wr = jnp.broadcast_to(w, (_M, _M))
    wc = wr.T
    i_idx = jax.lax.broadcasted_iota(jnp.int32, (_M, _M), 0)
    j_idx = jax.lax.broadcasted_iota(jnp.int32, (_M, _M), 1)
    gt = (wc > wr).astype(jnp.float32)
    tie = jnp.logical_and(wc == wr, i_idx < j_idx).astype(jnp.float32)
    rank = jnp.sum(gt + tie, axis=0, keepdims=True)
    keep = rank < float(_DS)
    wk = jnp.where(keep, w, 0.0)
    w_ref[...] = wk / jnp.sum(wk)


def _predictor_kernel(x_ref, w_ref, Wp0_ref, bp0_ref, Wp1_ref, bp1_ref, out_ref):
    xw = x_ref[...] * w_ref[...]
    h = jnp.maximum(
        jnp.dot(xw, Wp0_ref[...], preferred_element_type=jnp.float32)
        + bp0_ref[...], 0.0)
    out_ref[...] = (
        jnp.dot(h, Wp1_ref[...], preferred_element_type=jnp.float32)
        + bp1_ref[...])


def _forward(x, Ws0, bs0, Ws1, bs1, Wp0, bp0, Wp1, bp1):
    B, M = x.shape
    H = Ws0.shape[1]
    nt = B // _BT

    bs0_2d = bs0.reshape(1, H)
    bs1_2d = bs1.reshape(1, M)
    bp0_2d = bp0.reshape(1, H)
    bp1_2d = bp1.reshape(1, M)

    parts = pl.pallas_call(
        _selector_kernel,
        grid=(nt,),
        in_specs=[
            pl.BlockSpec((_BT, M), lambda t: (t, 0)),
            pl.BlockSpec((M, H), lambda t: (0, 0)),
            pl.BlockSpec((1, H), lambda t: (0, 0)),
            pl.BlockSpec((H, M), lambda t: (0, 0)),
            pl.BlockSpec((1, M), lambda t: (0, 0)),
        ],
        out_specs=pl.BlockSpec((1, 1, M), lambda t: (t, 0, 0)),
        out_shape=jax.ShapeDtypeStruct((nt, 1, M), jnp.float32),
        compiler_params=pltpu.CompilerParams(
            dimension_semantics=("parallel",)),
    )(x, Ws0, bs0_2d, Ws1, bs1_2d)

    parts = jax.lax.psum(parts, "b")

    w = pl.pallas_call(
        _finalize_kernel,
        in_specs=[pl.BlockSpec((nt, 1, M), lambda: (0, 0, 0))],
        out_specs=pl.BlockSpec((1, M), lambda: (0, 0)),
        out_shape=jax.ShapeDtypeStruct((1, M), jnp.float32),
    )(parts)

    out = pl.pallas_call(
        _predictor_kernel,
        grid=(nt,),
        in_specs=[
            pl.BlockSpec((_BT, M), lambda t: (t, 0)),
            pl.BlockSpec((1, M), lambda t: (0, 0)),
            pl.BlockSpec((M, H), lambda t: (0, 0)),
            pl.BlockSpec((1, H), lambda t: (0, 0)),
            pl.BlockSpec((H, M), lambda t: (0, 0)),
            pl.BlockSpec((1, M), lambda t: (0, 0)),
        ],
        out_specs=pl.BlockSpec((_BT, M), lambda t: (t, 0)),
        out_shape=jax.ShapeDtypeStruct((B, M), jnp.float32),
        compiler_params=pltpu.CompilerParams(
            dimension_semantics=("parallel",)),
    )(x, w, Wp0, bp0_2d, Wp1, bp1_2d)

    return out


def kernel(x, Ws0, bs0, Ws1, bs1, Wp0, bp0, Wp1, bp1):
    ndev = max(1, min(2, len(jax.devices())))
    mesh = Mesh(np.array(jax.devices()[:ndev]), ("b",))
    rep = P()
    fn = _shard_map(
        _forward,
        mesh=mesh,
        in_specs=(P("b"), rep, rep, rep, rep, rep, rep, rep, rep),
        out_specs=P("b"),
        check_rep=False,
    )
    return fn(x, Ws0, bs0, Ws1, bs1, Wp0, bp0, Wp1, bp1)


# speedup vs baseline: 6.7305x; 6.7305x over previous
import jax
import jax.numpy as jnp
from jax.experimental import pallas as pl
from jax.experimental.pallas import tpu as pltpu

_M = 512
_DS = 256
_BT = 1024


def _fused_kernel(x_ref, Ws0_ref, bs0_ref, Ws1_ref, bs1_ref,
                  Wp0_ref, bp0_ref, Wp1_ref, bp1_ref, out_ref, w_ref):
    i = pl.program_id(0)
    nt = (pl.num_programs(0) - 1) // 2

    @pl.when(i < nt)
    def _selector():
        x = x_ref[...]
        h = jnp.maximum(
            jnp.dot(x, Ws0_ref[...], preferred_element_type=jnp.float32)
            + bs0_ref[...], 0.0)
        s = (jnp.dot(h, Ws1_ref[...], preferred_element_type=jnp.float32)
             + bs1_ref[...])
        e = jnp.exp(-s)
        p = e / jnp.sum(e, axis=1, keepdims=True)
        part = jnp.sum(p, axis=0, keepdims=True)

        @pl.when(i == 0)
        def _():
            w_ref[...] = part

        @pl.when(i != 0)
        def _():
            w_ref[...] = w_ref[...] + part

    @pl.when(i == nt)
    def _finalize():
        w = w_ref[...]
        wr = jnp.broadcast_to(w, (_M, _M))
        wc = wr.T
        i_idx = jax.lax.broadcasted_iota(jnp.int32, (_M, _M), 0)
        j_idx = jax.lax.broadcasted_iota(jnp.int32, (_M, _M), 1)
        gt = (wc > wr).astype(jnp.float32)
        tie = jnp.logical_and(wc == wr, i_idx < j_idx).astype(jnp.float32)
        rank = jnp.sum(gt + tie, axis=0, keepdims=True)
        keep = rank < float(_DS)
        wk = jnp.where(keep, w, 0.0)
        w_ref[...] = wk / jnp.sum(wk)

    @pl.when(i > nt)
    def _predictor():
        xw = x_ref[...] * w_ref[...]
        h = jnp.maximum(
            jnp.dot(xw, Wp0_ref[...], preferred_element_type=jnp.float32)
            + bp0_ref[...], 0.0)
        out_ref[...] = (
            jnp.dot(h, Wp1_ref[...], preferred_element_type=jnp.float32)
            + bp1_ref[...])


def kernel(x, Ws0, bs0, Ws1, bs1, Wp0, bp0, Wp1, bp1):
    B, M = x.shape
    H = Ws0.shape[1]
    nt = B // _BT

    bs0_2d = bs0.reshape(1, H)
    bs1_2d = bs1.reshape(1, M)
    bp0_2d = bp0.reshape(1, H)
    bp1_2d = bp1.reshape(1, M)

    def x_map(i):
        return (jnp.where(i < nt, i, jnp.maximum(i - nt - 1, 0)), 0)

    def out_map(i):
        return (jnp.maximum(i - nt - 1, 0), 0)

    full = lambda i: (0, 0)

    out = pl.pallas_call(
        _fused_kernel,
        grid=(2 * nt + 1,),
        in_specs=[
            pl.BlockSpec((_BT, M), x_map),
            pl.BlockSpec((M, H), full),
            pl.BlockSpec((1, H), full),
            pl.BlockSpec((H, M), full),
            pl.BlockSpec((1, M), full),
            pl.BlockSpec((M, H), full),
            pl.BlockSpec((1, H), full),
            pl.BlockSpec((H, M), full),
            pl.BlockSpec((1, M), full),
        ],
        out_specs=pl.BlockSpec((_BT, M), out_map),
        out_shape=jax.ShapeDtypeStruct((B, M), jnp.float32),
        scratch_shapes=[pltpu.VMEM((1, M), jnp.float32)],
        compiler_params=pltpu.CompilerParams(
            dimension_semantics=("arbitrary",)),
    )(x, Ws0, bs0_2d, Ws1, bs1_2d, Wp0, bp0_2d, Wp1, bp1_2d)

    return out


# speedup vs baseline: 6.8746x; 1.0214x over previous
import jax
import jax.numpy as jnp
from jax.experimental import pallas as pl
from jax.experimental.pallas import tpu as pltpu

_M = 512
_DS = 256
_BT = 2048


def _fused_kernel(x_ref, Ws0_ref, bs0_ref, Ws1_ref, bs1_ref,
                  Wp0_ref, bp0_ref, Wp1_ref, bp1_ref, out_ref, w_ref):
    i = pl.program_id(0)
    nt = (pl.num_programs(0) - 1) // 2

    @pl.when(i < nt)
    def _selector():
        x = x_ref[...]
        h = jnp.maximum(
            jnp.dot(x, Ws0_ref[...], preferred_element_type=jnp.float32)
            + bs0_ref[...], 0.0)
        s = (jnp.dot(h, Ws1_ref[...], preferred_element_type=jnp.float32)
             + bs1_ref[...])
        e = jnp.exp(-s)
        p = e / jnp.sum(e, axis=1, keepdims=True)
        part = jnp.sum(p, axis=0, keepdims=True)

        @pl.when(i == 0)
        def _():
            w_ref[...] = part

        @pl.when(i != 0)
        def _():
            w_ref[...] = w_ref[...] + part

    @pl.when(i == nt)
    def _finalize():
        w = w_ref[...]
        wr = jnp.broadcast_to(w, (_M, _M))
        wc = wr.T
        i_idx = jax.lax.broadcasted_iota(jnp.int32, (_M, _M), 0)
        j_idx = jax.lax.broadcasted_iota(jnp.int32, (_M, _M), 1)
        gt = (wc > wr).astype(jnp.float32)
        tie = jnp.logical_and(wc == wr, i_idx < j_idx).astype(jnp.float32)
        rank = jnp.sum(gt + tie, axis=0, keepdims=True)
        keep = rank < float(_DS)
        wk = jnp.where(keep, w, 0.0)
        w_ref[...] = wk / jnp.sum(wk)

    @pl.when(i > nt)
    def _predictor():
        xw = x_ref[...] * w_ref[...]
        h = jnp.maximum(
            jnp.dot(xw, Wp0_ref[...], preferred_element_type=jnp.float32)
            + bp0_ref[...], 0.0)
        out_ref[...] = (
            jnp.dot(h, Wp1_ref[...], preferred_element_type=jnp.float32)
            + bp1_ref[...])


def kernel(x, Ws0, bs0, Ws1, bs1, Wp0, bp0, Wp1, bp1):
    B, M = x.shape
    H = Ws0.shape[1]
    nt = B // _BT

    bs0_2d = bs0.reshape(1, H)
    bs1_2d = bs1.reshape(1, M)
    bp0_2d = bp0.reshape(1, H)
    bp1_2d = bp1.reshape(1, M)

    def x_map(i):
        return (jnp.where(i < nt, i, jnp.maximum(i - nt - 1, 0)), 0)

    def out_map(i):
        return (jnp.maximum(i - nt - 1, 0), 0)

    full = lambda i: (0, 0)

    out = pl.pallas_call(
        _fused_kernel,
        grid=(2 * nt + 1,),
        in_specs=[
            pl.BlockSpec((_BT, M), x_map),
            pl.BlockSpec((M, H), full),
            pl.BlockSpec((1, H), full),
            pl.BlockSpec((H, M), full),
            pl.BlockSpec((1, M), full),
            pl.BlockSpec((M, H), full),
            pl.BlockSpec((1, H), full),
            pl.BlockSpec((H, M), full),
            pl.BlockSpec((1, M), full),
        ],
        out_specs=pl.BlockSpec((_BT, M), out_map),
        out_shape=jax.ShapeDtypeStruct((B, M), jnp.float32),
        scratch_shapes=[pltpu.VMEM((1, M), jnp.float32)],
        compiler_params=pltpu.CompilerParams(
            dimension_semantics=("arbitrary",)),
    )(x, Ws0, bs0_2d, Ws1, bs1_2d, Wp0, bp0_2d, Wp1, bp1_2d)

    return out


# speedup vs baseline: 6.8941x; 1.0028x over previous
import jax
import jax.numpy as jnp
from jax.experimental import pallas as pl
from jax.experimental.pallas import tpu as pltpu

_M = 512
_DS = 256
_BT = 4096


def _fused_kernel(x_ref, Ws0_ref, bs0_ref, Ws1_ref, bs1_ref,
                  Wp0_ref, bp0_ref, Wp1_ref, bp1_ref, out_ref, w_ref):
    i = pl.program_id(0)
    nt = (pl.num_programs(0) - 1) // 2

    @pl.when(i < nt)
    def _selector():
        x = x_ref[...]
        h = jnp.maximum(
            jnp.dot(x, Ws0_ref[...], preferred_element_type=jnp.float32)
            + bs0_ref[...], 0.0)
        s = (jnp.dot(h, Ws1_ref[...], preferred_element_type=jnp.float32)
             + bs1_ref[...])
        e = jnp.exp(-s)
        p = e / jnp.sum(e, axis=1, keepdims=True)
        part = jnp.sum(p, axis=0, keepdims=True)

        @pl.when(i == 0)
        def _():
            w_ref[...] = part

        @pl.when(i != 0)
        def _():
            w_ref[...] = w_ref[...] + part

    @pl.when(i == nt)
    def _finalize():
        w = w_ref[...]
        wr = jnp.broadcast_to(w, (_M, _M))
        wc = wr.T
        i_idx = jax.lax.broadcasted_iota(jnp.int32, (_M, _M), 0)
        j_idx = jax.lax.broadcasted_iota(jnp.int32, (_M, _M), 1)
        gt = (wc > wr).astype(jnp.float32)
        tie = jnp.logical_and(wc == wr, i_idx < j_idx).astype(jnp.float32)
        rank = jnp.sum(gt + tie, axis=0, keepdims=True)
        keep = rank < float(_DS)
        wk = jnp.where(keep, w, 0.0)
        w_ref[...] = wk / jnp.sum(wk)

    @pl.when(i > nt)
    def _predictor():
        xw = x_ref[...] * w_ref[...]
        h = jnp.maximum(
            jnp.dot(xw, Wp0_ref[...], preferred_element_type=jnp.float32)
            + bp0_ref[...], 0.0)
        out_ref[...] = (
            jnp.dot(h, Wp1_ref[...], preferred_element_type=jnp.float32)
            + bp1_ref[...])


def kernel(x, Ws0, bs0, Ws1, bs1, Wp0, bp0, Wp1, bp1):
    B, M = x.shape
    H = Ws0.shape[1]
    nt = B // _BT

    bs0_2d = bs0.reshape(1, H)
    bs1_2d = bs1.reshape(1, M)
    bp0_2d = bp0.reshape(1, H)
    bp1_2d = bp1.reshape(1, M)

    def x_map(i):
        return (jnp.where(i < nt, i, jnp.maximum(i - nt - 1, 0)), 0)

    def out_map(i):
        return (jnp.maximum(i - nt - 1, 0), 0)

    full = lambda i: (0, 0)

    out = pl.pallas_call(
        _fused_kernel,
        grid=(2 * nt + 1,),
        in_specs=[
            pl.BlockSpec((_BT, M), x_map),
            pl.BlockSpec((M, H), full),
            pl.BlockSpec((1, H), full),
            pl.BlockSpec((H, M), full),
            pl.BlockSpec((1, M), full),
            pl.BlockSpec((M, H), full),
            pl.BlockSpec((1, H), full),
            pl.BlockSpec((H, M), full),
            pl.BlockSpec((1, M), full),
        ],
        out_specs=pl.BlockSpec((_BT, M), out_map),
        out_shape=jax.ShapeDtypeStruct((B, M), jnp.float32),
        scratch_shapes=[pltpu.VMEM((1, M), jnp.float32)],
        compiler_params=pltpu.CompilerParams(
            dimension_semantics=("arbitrary",)),
    )(x, Ws0, bs0_2d, Ws1, bs1_2d, Wp0, bp0_2d, Wp1, bp1_2d)

    return out
